# baseline (speedup 1.0000x reference)
import functools

import jax
import jax.numpy as jnp
from jax import lax
from jax.experimental import pallas as pl
from jax.experimental.pallas import tpu as pltpu
from jax.experimental.pallas import tpu_sc as plsc

EPS = 1e-5

_NC = 2
_NS = 16
_NW = _NC * _NS


def _make_gather(B, dims):
    bpw = B // _NW
    T = len(dims)
    D = sum(dims)
    feat = [(t, c) for t, d in enumerate(dims) for c in range(d)]

    mesh = plsc.VectorSubcoreMesh(core_axis_name="c", subcore_axis_name="s")

    @functools.partial(
        pl.kernel,
        mesh=mesh,
        out_type=jax.ShapeDtypeStruct((D, B), jnp.float32),
        scratch_types=[
            pltpu.VMEM((T, bpw), jnp.int32),
            pltpu.VMEM((D, bpw), jnp.float32),
            pltpu.SemaphoreType.DMA,
        ],
        compiler_params=pltpu.CompilerParams(
            use_tc_tiling_on_sc=False, needs_layout_passes=False
        ),
    )
    def gather_k(*refs):
        i_hbms = refs[:T]
        tables = refs[T:2 * T]
        out_hbm = refs[2 * T]
        idxb_v, xt_v, sem = refs[2 * T + 1:]
        wid = lax.axis_index("s") * _NC + lax.axis_index("c")
        base = wid * bpw
        for s, i_hbm in enumerate(i_hbms):
            pltpu.sync_copy(i_hbm.at[pl.ds(base, bpw)], idxb_v.at[s])

        copies = [
            pltpu.async_copy(
                tables[tid].at[c_local].at[idxb_v.at[tid]], xt_v.at[c], sem)
            for c, (tid, c_local) in enumerate(feat)
        ]
        for cp in copies:
            cp.wait()
        pltpu.sync_copy(xt_v, out_hbm.at[:, pl.ds(base, bpw)])

    return gather_k


def _mlp_body(xt, xt2, g0, b0, w1, b1, g1, be1, w2, b2, g2, be2, wo, bo, out):
    n = out.shape[0] * out.shape[1]

    def stats(h):
        m = (jnp.sum(jnp.sum(h, axis=2), axis=1) / n)
        d = h - m[:, None, None]
        v = (jnp.sum(jnp.sum(d * d, axis=2), axis=1) / n)
        s = lax.rsqrt(v + EPS)
        return m, s

    def mm(w, h):
        return lax.dot_general(
            w, h, dimension_numbers=(((1,), (0,)), ((), ())),
            preferred_element_type=jnp.float32)

    x = jnp.concatenate([xt[...], xt2[...]], axis=0)
    m, s = stats(x)
    a = g0[...] * s
    w1e = w1[...] * a[None, :]
    b1e = mm(w1[...], (b0[...] - m * a)[:, None]) + b1[...][:, None]
    h = jnp.maximum(mm(w1e, x) + b1e[:, :, None], 0.0)

    m, s = stats(h)
    a = g1[...] * s
    w2e = w2[...] * a[None, :]
    b2e = mm(w2[...], (be1[...] - m * a)[:, None]) + b2[...][:, None]
    h = jnp.maximum(mm(w2e, h) + b2e[:, :, None], 0.0)

    m, s = stats(h)
    a = g2[...] * s
    woe = (wo[...][0] * a)[:, None, None]
    boe = jnp.sum(wo[...][0] * (be2[...] - m * a)) + bo[...]
    out[...] = jnp.sum(h * woe, axis=0) + boe[:, None]


def kernel(input, item_emb, cat_emb, shop_emb, g0, b0, W1, b1, g1, be1,
           W2, b2, g2, be2, Wo, bo):
    B = input.shape[0]
    V, D1 = item_emb.shape
    D2 = cat_emb.shape[1]
    D3 = shop_emb.shape[1]
    D = D1 + D2 + D3

    idx = input.astype(jnp.int32)
    i0 = idx[:, 0]
    i1 = idx[:, 1]
    i2 = idx[:, 2]

    gather_cs = _make_gather(B, (D2, D3))
    xt_cs = gather_cs(i1, i2, cat_emb.T, shop_emb.T)
    gather_it = _make_gather(B, (D1,))
    xt_it = gather_it(i0, item_emb.T)

    xt_it3 = xt_it.reshape(D1, 128, B // 128)
    xt_cs3 = xt_cs.reshape(D2 + D3, 128, B // 128)

    mlp = pl.pallas_call(
        _mlp_body,
        out_shape=jax.ShapeDtypeStruct((128, B // 128), jnp.float32),
    )
    out = mlp(xt_it3, xt_cs3, g0, b0, W1, b1, g1, be1, W2, b2, g2, be2,
              Wo, bo)
    return out.reshape(B)
